# baseline (speedup 1.0000x reference)
import functools

import jax
import jax.numpy as jnp
from jax import lax
from jax.experimental import pallas as pl
from jax.experimental.pallas import tpu as pltpu
from jax.experimental.pallas import tpu_sc as plsc

B, N2, N1 = 16, 4096, 1024
C1, C2 = 256, 128
OUT1, OUT2 = 256, 256
CIN = C1 + C2
TQ = 2048
NQ = B * N2
NPTS = float(NQ)
NSPLIT = 8
BH = B // NSPLIT

SC_NC = 2
SC_NS = 16
SC_NW = SC_NC * SC_NS
CH = 32


def _knn_body(xyz1_ref, xyz2_ref, w_ref, i_ref):
    b = pl.program_id(0)
    p1 = xyz1_ref[...]
    p2 = xyz2_ref[...]
    p1p = jnp.concatenate([p1, jnp.zeros((5, N1), jnp.float32)], axis=0)
    p1t = jnp.transpose(p1p)

    acc = None
    for d in range(3):
        diff = p1t[:, d:d + 1] - p2[d:d + 1, :]
        sq = diff * diff
        acc = sq if acc is None else acc + sq

    iota = lax.broadcasted_iota(jnp.int32, (N1, TQ), 0)
    dists = []
    idxs = []
    for _ in range(3):
        m = jnp.min(acc, axis=0, keepdims=True)
        im = jnp.min(jnp.where(acc == m, iota, N1), axis=0,
                     keepdims=True)
        acc = jnp.where(iota == im, jnp.float32(jnp.inf), acc)
        dists.append(m)
        idxs.append(im)

    invs = [1.0 / jnp.where(d < 1e-10, jnp.float32(1e-10), d) for d in dists]
    norm = (invs[0] + invs[1]) + invs[2]
    ws = [v / norm for v in invs]

    w_ref[...] = jnp.concatenate(ws + [jnp.zeros((5, TQ), jnp.float32)],
                                 axis=0)
    base = b * N1
    iflat = [ix + base for ix in idxs]
    i_ref[...] = jnp.concatenate(iflat + [jnp.zeros((5, TQ), jnp.int32)],
                                 axis=0)


def _knn(xyz1, xyz2, nb):
    return pl.pallas_call(
        _knn_body,
        grid=(nb, N2 // TQ),
        in_specs=[
            pl.BlockSpec((None, 3, N1), lambda b, t: (b, 0, 0)),
            pl.BlockSpec((None, 3, TQ), lambda b, t: (b, 0, t)),
        ],
        out_specs=[
            pl.BlockSpec((None, 8, TQ), lambda b, t: (b, 0, t)),
            pl.BlockSpec((None, 8, TQ), lambda b, t: (b, 0, t)),
        ],
        out_shape=[
            jax.ShapeDtypeStruct((nb, 8, N2), jnp.float32),
            jax.ShapeDtypeStruct((nb, 8, N2), jnp.int32),
        ],
    )(xyz1, xyz2)


def _make_interp_body(qpw):
    nch = qpw // CH
    npair = nch // 2

    def body(idx_hbm, w_hbm, feat_hbm, out_hbm,
             i0v, i1v, i2v, w0v, w1v, w2v,
             r0a, r1a, r2a, r0b, r1b, r2b, ov, sema, semb):
        wid = lax.axis_index("s") * SC_NC + lax.axis_index("c")
        q0 = wid * qpw
        b = q0 // N2
        n0 = q0 - b * N2
        pltpu.sync_copy(idx_hbm.at[pl.ds((b * 8 + 0) * N2 + n0, qpw)], i0v)
        pltpu.sync_copy(idx_hbm.at[pl.ds((b * 8 + 1) * N2 + n0, qpw)], i1v)
        pltpu.sync_copy(idx_hbm.at[pl.ds((b * 8 + 2) * N2 + n0, qpw)], i2v)
        pltpu.sync_copy(w_hbm.at[pl.ds((b * 8 + 0) * N2 + n0, qpw)], w0v)
        pltpu.sync_copy(w_hbm.at[pl.ds((b * 8 + 1) * N2 + n0, qpw)], w1v)
        pltpu.sync_copy(w_hbm.at[pl.ds((b * 8 + 2) * N2 + n0, qpw)], w2v)

        def issue(c, r0, r1, r2, sem):
            s = pl.ds(c * CH, CH)
            pltpu.async_copy(feat_hbm.at[i0v.at[s]], r0, sem)
            pltpu.async_copy(feat_hbm.at[i1v.at[s]], r1, sem)
            pltpu.async_copy(feat_hbm.at[i2v.at[s]], r2, sem)

        def drain(r0, r1, r2, sem):
            dummy = feat_hbm.at[pl.ds(0, CH)]
            pltpu.make_async_copy(dummy, r0, sem).wait()
            pltpu.make_async_copy(dummy, r1, sem).wait()
            pltpu.make_async_copy(dummy, r2, sem).wait()

        def compute(c, r0, r1, r2):
            def group(g, _):
                o = c * CH + g * 16
                wv0 = w0v[pl.ds(o, 16)]
                wv1 = w1v[pl.ds(o, 16)]
                wv2 = w2v[pl.ds(o, 16)]
                for qi in range(16):
                    q = g * 16 + qi
                    w0 = wv0[qi]
                    w1 = wv1[qi]
                    w2 = wv2[qi]
                    for j in range(C1 // 16):
                        s = pl.ds(j * 16, 16)
                        ov[q, s] = ((w0 * r0[q, s] + w1 * r1[q, s])
                                    + w2 * r2[q, s])
                return 0

            lax.fori_loop(0, CH // 16, group, 0)
            pltpu.sync_copy(ov, out_hbm.at[pl.ds(q0 + c * CH, CH)])

        issue(0, r0a, r1a, r2a, sema)

        def pair(p, _):
            c0 = 2 * p
            issue(c0 + 1, r0b, r1b, r2b, semb)
            drain(r0a, r1a, r2a, sema)
            compute(c0, r0a, r1a, r2a)

            @pl.when(p < npair - 1)
            def _():
                issue(c0 + 2, r0a, r1a, r2a, sema)

            drain(r0b, r1b, r2b, semb)
            compute(c0 + 1, r0b, r1b, r2b)
            return 0

        lax.fori_loop(0, npair, pair, 0)

    return body


def _interp(idx_flat, w_flat, feat_flat, nb):
    nqh = nb * N2
    qpw = nqh // SC_NW
    mesh = plsc.VectorSubcoreMesh(core_axis_name="c", subcore_axis_name="s")
    f = functools.partial(
        pl.kernel,
        out_type=jax.ShapeDtypeStruct((nqh, C1), jnp.float32),
        mesh=mesh,
        scratch_types=[
            pltpu.VMEM((qpw,), jnp.int32),
            pltpu.VMEM((qpw,), jnp.int32),
            pltpu.VMEM((qpw,), jnp.int32),
            pltpu.VMEM((qpw,), jnp.float32),
            pltpu.VMEM((qpw,), jnp.float32),
            pltpu.VMEM((qpw,), jnp.float32),
            pltpu.VMEM((CH, C1), jnp.float32),
            pltpu.VMEM((CH, C1), jnp.float32),
            pltpu.VMEM((CH, C1), jnp.float32),
            pltpu.VMEM((CH, C1), jnp.float32),
            pltpu.VMEM((CH, C1), jnp.float32),
            pltpu.VMEM((CH, C1), jnp.float32),
            pltpu.VMEM((CH, C1), jnp.float32),
            pltpu.SemaphoreType.DMA,
            pltpu.SemaphoreType.DMA,
        ],
    )(_make_interp_body(qpw))
    return f(idx_flat, w_flat, feat_flat)


def _conv1_body(nf_ref, f2_ref, w1_ref, b1_ref, y_ref, st_ref):
    nf = nf_ref[...]
    f2 = f2_ref[...]
    w = w1_ref[...]
    ya = lax.dot_general(nf.astype(jnp.bfloat16),
                         w[:, :C1].astype(jnp.bfloat16),
                         (((1,), (1,)), ((), ())),
                         preferred_element_type=jnp.float32)
    yb = lax.dot_general(f2.astype(jnp.bfloat16),
                         w[:, C1:].astype(jnp.bfloat16),
                         (((0,), (1,)), ((), ())),
                         preferred_element_type=jnp.float32)
    y = ya + yb + b1_ref[...]
    y_ref[...] = y

    first = (pl.program_id(0) == 0) & (pl.program_id(1) == 0)

    @pl.when(first)
    def _():
        st_ref[...] = jnp.zeros((8, OUT1), jnp.float32)

    s = jnp.sum(y, axis=0, keepdims=True)
    sq = jnp.sum(y * y, axis=0, keepdims=True)
    st_ref[...] += jnp.concatenate(
        [s, sq, jnp.zeros((6, OUT1), jnp.float32)], axis=0)


def _conv1(nf, feat2, W1, b1r, nb):
    return pl.pallas_call(
        _conv1_body,
        grid=(nb, N2 // TQ),
        in_specs=[
            pl.BlockSpec((None, TQ, C1), lambda b, t: (b, t, 0)),
            pl.BlockSpec((None, C2, TQ), lambda b, t: (b, 0, t)),
            pl.BlockSpec((OUT1, CIN), lambda b, t: (0, 0)),
            pl.BlockSpec((1, OUT1), lambda b, t: (0, 0)),
        ],
        out_specs=[
            pl.BlockSpec((None, TQ, OUT1), lambda b, t: (b, t, 0)),
            pl.BlockSpec((8, OUT1), lambda b, t: (0, 0)),
        ],
        out_shape=[
            jax.ShapeDtypeStruct((nb, N2, OUT1), jnp.float32),
            jax.ShapeDtypeStruct((8, OUT1), jnp.float32),
        ],
    )(nf, feat2, W1, b1r)


def _make_conv2_body(nstats):
    def body(*refs):
        y1_ref = refs[0]
        sts = refs[1:1 + nstats]
        g1_ref, be1_ref, w2_ref, b2_ref, y_ref, st_ref = refs[1 + nstats:]
        st = sts[0][...]
        for r in sts[1:]:
            st = st + r[...]
        mu = st[0:1, :] / NPTS
        var = st[1:2, :] / NPTS - mu * mu
        scale = g1_ref[...] / jnp.sqrt(var + 1e-3)
        shift = be1_ref[...] - mu * scale
        h = jnp.maximum(y1_ref[...] * scale + shift, 0.0)
        y = lax.dot_general(h.astype(jnp.bfloat16),
                            w2_ref[...].astype(jnp.bfloat16),
                            (((1,), (1,)), ((), ())),
                            preferred_element_type=jnp.float32) + b2_ref[...]
        y_ref[...] = y

        first = (pl.program_id(0) == 0) & (pl.program_id(1) == 0)

        @pl.when(first)
        def _():
            st_ref[...] = jnp.zeros((8, OUT2), jnp.float32)

        sm = jnp.sum(y, axis=0, keepdims=True)
        sq = jnp.sum(y * y, axis=0, keepdims=True)
        st_ref[...] += jnp.concatenate(
            [sm, sq, jnp.zeros((6, OUT2), jnp.float32)], axis=0)

    return body


def _conv2(y1, sts, g1r, be1r, W2, b2r, nb):
    return pl.pallas_call(
        _make_conv2_body(len(sts)),
        grid=(nb, N2 // TQ),
        in_specs=[pl.BlockSpec((None, TQ, OUT1), lambda b, t: (b, t, 0))]
        + [pl.BlockSpec((8, OUT1), lambda b, t: (0, 0)) for _ in sts]
        + [
            pl.BlockSpec((1, OUT1), lambda b, t: (0, 0)),
            pl.BlockSpec((1, OUT1), lambda b, t: (0, 0)),
            pl.BlockSpec((OUT2, OUT1), lambda b, t: (0, 0)),
            pl.BlockSpec((1, OUT2), lambda b, t: (0, 0)),
        ],
        out_specs=[
            pl.BlockSpec((None, TQ, OUT2), lambda b, t: (b, t, 0)),
            pl.BlockSpec((8, OUT2), lambda b, t: (0, 0)),
        ],
        out_shape=[
            jax.ShapeDtypeStruct((nb, N2, OUT2), jnp.float32),
            jax.ShapeDtypeStruct((8, OUT2), jnp.float32),
        ],
    )(y1, *sts, g1r, be1r, W2, b2r)


def _make_final_body(nstats):
    def body(*refs):
        y2_ref = refs[0]
        sts = refs[1:1 + nstats]
        g2_ref, be2_ref, o_ref = refs[1 + nstats:]
        st = sts[0][...]
        for r in sts[1:]:
            st = st + r[...]
        mu = st[0:1, :] / NPTS
        var = st[1:2, :] / NPTS - mu * mu
        scale = g2_ref[...] / jnp.sqrt(var + 1e-3)
        shift = be2_ref[...] - mu * scale
        o = jnp.maximum(y2_ref[...] * scale + shift, 0.0)
        o_ref[...] = jnp.transpose(o)

    return body


def _final(y2, sts, g2r, be2r, nb, bofs, prev):
    out_spec = pl.BlockSpec((None, OUT2, TQ), lambda b, t: (b + bofs, 0, t))
    in_specs = [pl.BlockSpec((None, TQ, OUT2), lambda b, t: (b, t, 0))] \
        + [pl.BlockSpec((8, OUT2), lambda b, t: (0, 0)) for _ in sts] \
        + [
            pl.BlockSpec((1, OUT2), lambda b, t: (0, 0)),
            pl.BlockSpec((1, OUT2), lambda b, t: (0, 0)),
        ]
    body = _make_final_body(len(sts))
    if prev is None:
        return pl.pallas_call(
            body,
            grid=(nb, N2 // TQ),
            in_specs=in_specs,
            out_specs=out_spec,
            out_shape=jax.ShapeDtypeStruct((B, OUT2, N2), jnp.float32),
        )(y2, *sts, g2r, be2r)

    def body2(prev_ref, *refs):
        body(*refs)

    return pl.pallas_call(
        body2,
        grid=(nb, N2 // TQ),
        in_specs=[pl.BlockSpec(memory_space=pl.ANY)] + in_specs,
        out_specs=out_spec,
        out_shape=jax.ShapeDtypeStruct((B, OUT2, N2), jnp.float32),
        input_output_aliases={0: 0},
    )(prev, y2, *sts, g2r, be2r)


def kernel(xyz2, xyz1, feat2, feat1, W1, b1, g1, be1, W2, b2, g2, be2):
    b1r = b1.reshape(1, OUT1)
    g1r = g1.reshape(1, OUT1)
    be1r = be1.reshape(1, OUT1)
    b2r = b2.reshape(1, OUT2)
    g2r = g2.reshape(1, OUT2)
    be2r = be2.reshape(1, OUT2)

    nfs = []
    for h in range(NSPLIT):
        sl = slice(h * BH, (h + 1) * BH)
        wout, iout = _knn(xyz1[sl], xyz2[sl], BH)
        feat_flat = jnp.transpose(feat1[sl], (0, 2, 1)).reshape(BH * N1, C1)
        nf = _interp(iout.reshape(-1), wout.reshape(-1), feat_flat, BH)
        nfs.append(nf.reshape(BH, N2, C1))

    y1s, st1s = [], []
    for h in range(NSPLIT):
        sl = slice(h * BH, (h + 1) * BH)
        y1, st1 = _conv1(nfs[h], feat2[sl], W1, b1r, BH)
        y1s.append(y1)
        st1s.append(st1)

    y2s, st2s = [], []
    for h in range(NSPLIT):
        y2, st2 = _conv2(y1s[h], st1s, g1r, be1r, W2, b2r, BH)
        y2s.append(y2)
        st2s.append(st2)

    out = None
    for h in range(NSPLIT):
        out = _final(y2s[h], st2s, g2r, be2r, BH, h * BH, out)
    return out


# speedup vs baseline: 1.0626x; 1.0626x over previous
import functools

import jax
import jax.numpy as jnp
from jax import lax
from jax.experimental import pallas as pl
from jax.experimental.pallas import tpu as pltpu
from jax.experimental.pallas import tpu_sc as plsc

B, N2, N1 = 16, 4096, 1024
C1, C2 = 256, 128
OUT1, OUT2 = 256, 256
CIN = C1 + C2
TQ = 2048
NQ = B * N2
NPTS = float(NQ)
SLICES = (2, 2, 4, 4, 4)

SC_NC = 2
SC_NS = 16
SC_NW = SC_NC * SC_NS
CH = 32


def _knn_body(xyz1_ref, xyz2_ref, w_ref, i_ref):
    b = pl.program_id(0)
    p1 = xyz1_ref[...]
    p2 = xyz2_ref[...]
    p1p = jnp.concatenate([p1, jnp.zeros((5, N1), jnp.float32)], axis=0)
    p1t = jnp.transpose(p1p)

    acc = None
    for d in range(3):
        diff = p1t[:, d:d + 1] - p2[d:d + 1, :]
        sq = diff * diff
        acc = sq if acc is None else acc + sq

    iota = lax.broadcasted_iota(jnp.int32, (N1, TQ), 0)
    dists = []
    idxs = []
    for _ in range(3):
        m = jnp.min(acc, axis=0, keepdims=True)
        im = jnp.min(jnp.where(acc == m, iota, N1), axis=0,
                     keepdims=True)
        acc = jnp.where(iota == im, jnp.float32(jnp.inf), acc)
        dists.append(m)
        idxs.append(im)

    invs = [1.0 / jnp.where(d < 1e-10, jnp.float32(1e-10), d) for d in dists]
    norm = (invs[0] + invs[1]) + invs[2]
    ws = [v / norm for v in invs]

    w_ref[...] = jnp.concatenate(ws + [jnp.zeros((5, TQ), jnp.float32)],
                                 axis=0)
    base = b * N1
    iflat = [ix + base for ix in idxs]
    i_ref[...] = jnp.concatenate(iflat + [jnp.zeros((5, TQ), jnp.int32)],
                                 axis=0)


def _knn(xyz1, xyz2, nb):
    return pl.pallas_call(
        _knn_body,
        grid=(nb, N2 // TQ),
        in_specs=[
            pl.BlockSpec((None, 3, N1), lambda b, t: (b, 0, 0)),
            pl.BlockSpec((None, 3, TQ), lambda b, t: (b, 0, t)),
        ],
        out_specs=[
            pl.BlockSpec((None, 8, TQ), lambda b, t: (b, 0, t)),
            pl.BlockSpec((None, 8, TQ), lambda b, t: (b, 0, t)),
        ],
        out_shape=[
            jax.ShapeDtypeStruct((nb, 8, N2), jnp.float32),
            jax.ShapeDtypeStruct((nb, 8, N2), jnp.int32),
        ],
    )(xyz1, xyz2)


def _make_interp_body(qpw):
    nch = qpw // CH
    npair = nch // 2

    def body(idx_hbm, w_hbm, feat_hbm, out_hbm,
             i0v, i1v, i2v, w0v, w1v, w2v,
             r0a, r1a, r2a, r0b, r1b, r2b, ov, sema, semb):
        wid = lax.axis_index("s") * SC_NC + lax.axis_index("c")
        q0 = wid * qpw
        b = q0 // N2
        n0 = q0 - b * N2
        pltpu.sync_copy(idx_hbm.at[pl.ds((b * 8 + 0) * N2 + n0, qpw)], i0v)
        pltpu.sync_copy(idx_hbm.at[pl.ds((b * 8 + 1) * N2 + n0, qpw)], i1v)
        pltpu.sync_copy(idx_hbm.at[pl.ds((b * 8 + 2) * N2 + n0, qpw)], i2v)
        pltpu.sync_copy(w_hbm.at[pl.ds((b * 8 + 0) * N2 + n0, qpw)], w0v)
        pltpu.sync_copy(w_hbm.at[pl.ds((b * 8 + 1) * N2 + n0, qpw)], w1v)
        pltpu.sync_copy(w_hbm.at[pl.ds((b * 8 + 2) * N2 + n0, qpw)], w2v)

        def issue(c, r0, r1, r2, sem):
            s = pl.ds(c * CH, CH)
            pltpu.async_copy(feat_hbm.at[i0v.at[s]], r0, sem)
            pltpu.async_copy(feat_hbm.at[i1v.at[s]], r1, sem)
            pltpu.async_copy(feat_hbm.at[i2v.at[s]], r2, sem)

        def drain(r0, r1, r2, sem):
            dummy = feat_hbm.at[pl.ds(0, CH)]
            pltpu.make_async_copy(dummy, r0, sem).wait()
            pltpu.make_async_copy(dummy, r1, sem).wait()
            pltpu.make_async_copy(dummy, r2, sem).wait()

        def compute(c, r0, r1, r2):
            def group(g, _):
                o = c * CH + g * 16
                wv0 = w0v[pl.ds(o, 16)]
                wv1 = w1v[pl.ds(o, 16)]
                wv2 = w2v[pl.ds(o, 16)]
                for qi in range(16):
                    q = g * 16 + qi
                    w0 = wv0[qi]
                    w1 = wv1[qi]
                    w2 = wv2[qi]
                    for j in range(C1 // 16):
                        s = pl.ds(j * 16, 16)
                        ov[q, s] = ((w0 * r0[q, s] + w1 * r1[q, s])
                                    + w2 * r2[q, s])
                return 0

            lax.fori_loop(0, CH // 16, group, 0)
            pltpu.sync_copy(ov, out_hbm.at[pl.ds(q0 + c * CH, CH)])

        issue(0, r0a, r1a, r2a, sema)

        def pair(p, _):
            c0 = 2 * p
            issue(c0 + 1, r0b, r1b, r2b, semb)
            drain(r0a, r1a, r2a, sema)
            compute(c0, r0a, r1a, r2a)

            @pl.when(p < npair - 1)
            def _():
                issue(c0 + 2, r0a, r1a, r2a, sema)

            drain(r0b, r1b, r2b, semb)
            compute(c0 + 1, r0b, r1b, r2b)
            return 0

        lax.fori_loop(0, npair, pair, 0)

    return body


def _interp(idx_flat, w_flat, feat_flat, nb):
    nqh = nb * N2
    qpw = nqh // SC_NW
    mesh = plsc.VectorSubcoreMesh(core_axis_name="c", subcore_axis_name="s")
    f = functools.partial(
        pl.kernel,
        out_type=jax.ShapeDtypeStruct((nqh, C1), jnp.float32),
        mesh=mesh,
        scratch_types=[
            pltpu.VMEM((qpw,), jnp.int32),
            pltpu.VMEM((qpw,), jnp.int32),
            pltpu.VMEM((qpw,), jnp.int32),
            pltpu.VMEM((qpw,), jnp.float32),
            pltpu.VMEM((qpw,), jnp.float32),
            pltpu.VMEM((qpw,), jnp.float32),
            pltpu.VMEM((CH, C1), jnp.float32),
            pltpu.VMEM((CH, C1), jnp.float32),
            pltpu.VMEM((CH, C1), jnp.float32),
            pltpu.VMEM((CH, C1), jnp.float32),
            pltpu.VMEM((CH, C1), jnp.float32),
            pltpu.VMEM((CH, C1), jnp.float32),
            pltpu.VMEM((CH, C1), jnp.float32),
            pltpu.SemaphoreType.DMA,
            pltpu.SemaphoreType.DMA,
        ],
    )(_make_interp_body(qpw))
    return f(idx_flat, w_flat, feat_flat)


def _conv1_body(nf_ref, f2_ref, w1_ref, b1_ref, y_ref, st_ref):
    nf = nf_ref[...]
    f2 = f2_ref[...]
    w = w1_ref[...]
    ya = lax.dot_general(nf.astype(jnp.bfloat16),
                         w[:, :C1].astype(jnp.bfloat16),
                         (((1,), (1,)), ((), ())),
                         preferred_element_type=jnp.float32)
    yb = lax.dot_general(f2.astype(jnp.bfloat16),
                         w[:, C1:].astype(jnp.bfloat16),
                         (((0,), (1,)), ((), ())),
                         preferred_element_type=jnp.float32)
    y = ya + yb + b1_ref[...]
    y_ref[...] = y

    first = (pl.program_id(0) == 0) & (pl.program_id(1) == 0)

    @pl.when(first)
    def _():
        st_ref[...] = jnp.zeros((8, OUT1), jnp.float32)

    s = jnp.sum(y, axis=0, keepdims=True)
    sq = jnp.sum(y * y, axis=0, keepdims=True)
    st_ref[...] += jnp.concatenate(
        [s, sq, jnp.zeros((6, OUT1), jnp.float32)], axis=0)


def _conv1(nf, feat2, W1, b1r, nb):
    return pl.pallas_call(
        _conv1_body,
        grid=(nb, N2 // TQ),
        in_specs=[
            pl.BlockSpec((None, TQ, C1), lambda b, t: (b, t, 0)),
            pl.BlockSpec((None, C2, TQ), lambda b, t: (b, 0, t)),
            pl.BlockSpec((OUT1, CIN), lambda b, t: (0, 0)),
            pl.BlockSpec((1, OUT1), lambda b, t: (0, 0)),
        ],
        out_specs=[
            pl.BlockSpec((None, TQ, OUT1), lambda b, t: (b, t, 0)),
            pl.BlockSpec((8, OUT1), lambda b, t: (0, 0)),
        ],
        out_shape=[
            jax.ShapeDtypeStruct((nb, N2, OUT1), jnp.float32),
            jax.ShapeDtypeStruct((8, OUT1), jnp.float32),
        ],
    )(nf, feat2, W1, b1r)


def _make_conv2_body(nstats):
    def body(*refs):
        y1_ref = refs[0]
        sts = refs[1:1 + nstats]
        g1_ref, be1_ref, w2_ref, b2_ref, y_ref, st_ref = refs[1 + nstats:]
        st = sts[0][...]
        for r in sts[1:]:
            st = st + r[...]
        mu = st[0:1, :] / NPTS
        var = st[1:2, :] / NPTS - mu * mu
        scale = g1_ref[...] / jnp.sqrt(var + 1e-3)
        shift = be1_ref[...] - mu * scale
        h = jnp.maximum(y1_ref[...] * scale + shift, 0.0)
        y = lax.dot_general(h.astype(jnp.bfloat16),
                            w2_ref[...].astype(jnp.bfloat16),
                            (((1,), (1,)), ((), ())),
                            preferred_element_type=jnp.float32) + b2_ref[...]
        y_ref[...] = y

        first = (pl.program_id(0) == 0) & (pl.program_id(1) == 0)

        @pl.when(first)
        def _():
            st_ref[...] = jnp.zeros((8, OUT2), jnp.float32)

        sm = jnp.sum(y, axis=0, keepdims=True)
        sq = jnp.sum(y * y, axis=0, keepdims=True)
        st_ref[...] += jnp.concatenate(
            [sm, sq, jnp.zeros((6, OUT2), jnp.float32)], axis=0)

    return body


def _conv2(y1, sts, g1r, be1r, W2, b2r, nb):
    return pl.pallas_call(
        _make_conv2_body(len(sts)),
        grid=(nb, N2 // TQ),
        in_specs=[pl.BlockSpec((None, TQ, OUT1), lambda b, t: (b, t, 0))]
        + [pl.BlockSpec((8, OUT1), lambda b, t: (0, 0)) for _ in sts]
        + [
            pl.BlockSpec((1, OUT1), lambda b, t: (0, 0)),
            pl.BlockSpec((1, OUT1), lambda b, t: (0, 0)),
            pl.BlockSpec((OUT2, OUT1), lambda b, t: (0, 0)),
            pl.BlockSpec((1, OUT2), lambda b, t: (0, 0)),
        ],
        out_specs=[
            pl.BlockSpec((None, TQ, OUT2), lambda b, t: (b, t, 0)),
            pl.BlockSpec((8, OUT2), lambda b, t: (0, 0)),
        ],
        out_shape=[
            jax.ShapeDtypeStruct((nb, N2, OUT2), jnp.float32),
            jax.ShapeDtypeStruct((8, OUT2), jnp.float32),
        ],
    )(y1, *sts, g1r, be1r, W2, b2r)


def _make_final_body(nstats):
    def body(*refs):
        y2_ref = refs[0]
        sts = refs[1:1 + nstats]
        g2_ref, be2_ref, o_ref = refs[1 + nstats:]
        st = sts[0][...]
        for r in sts[1:]:
            st = st + r[...]
        mu = st[0:1, :] / NPTS
        var = st[1:2, :] / NPTS - mu * mu
        scale = g2_ref[...] / jnp.sqrt(var + 1e-3)
        shift = be2_ref[...] - mu * scale
        o = jnp.maximum(y2_ref[...] * scale + shift, 0.0)
        o_ref[...] = jnp.transpose(o)

    return body


def _final(y2, sts, g2r, be2r, nb, bofs, prev):
    out_spec = pl.BlockSpec((None, OUT2, TQ), lambda b, t: (b + bofs, 0, t))
    in_specs = [pl.BlockSpec((None, TQ, OUT2), lambda b, t: (b, t, 0))] \
        + [pl.BlockSpec((8, OUT2), lambda b, t: (0, 0)) for _ in sts] \
        + [
            pl.BlockSpec((1, OUT2), lambda b, t: (0, 0)),
            pl.BlockSpec((1, OUT2), lambda b, t: (0, 0)),
        ]
    body = _make_final_body(len(sts))
    if prev is None:
        return pl.pallas_call(
            body,
            grid=(nb, N2 // TQ),
            in_specs=in_specs,
            out_specs=out_spec,
            out_shape=jax.ShapeDtypeStruct((B, OUT2, N2), jnp.float32),
        )(y2, *sts, g2r, be2r)

    def body2(prev_ref, *refs):
        body(*refs)

    return pl.pallas_call(
        body2,
        grid=(nb, N2 // TQ),
        in_specs=[pl.BlockSpec(memory_space=pl.ANY)] + in_specs,
        out_specs=out_spec,
        out_shape=jax.ShapeDtypeStruct((B, OUT2, N2), jnp.float32),
        input_output_aliases={0: 0},
    )(prev, y2, *sts, g2r, be2r)


def kernel(xyz2, xyz1, feat2, feat1, W1, b1, g1, be1, W2, b2, g2, be2):
    b1r = b1.reshape(1, OUT1)
    g1r = g1.reshape(1, OUT1)
    be1r = be1.reshape(1, OUT1)
    b2r = b2.reshape(1, OUT2)
    g2r = g2.reshape(1, OUT2)
    be2r = be2.reshape(1, OUT2)

    nfs = []
    offs = []
    o = 0
    for nb in SLICES:
        offs.append(o)
        sl = slice(o, o + nb)
        wout, iout = _knn(xyz1[sl], xyz2[sl], nb)
        feat_flat = jnp.transpose(feat1[sl], (0, 2, 1)).reshape(nb * N1, C1)
        nf = _interp(iout.reshape(-1), wout.reshape(-1), feat_flat, nb)
        nfs.append(nf.reshape(nb, N2, C1))
        o += nb

    y1s, st1s = [], []
    for h, nb in enumerate(SLICES):
        sl = slice(offs[h], offs[h] + nb)
        y1, st1 = _conv1(nfs[h], feat2[sl], W1, b1r, nb)
        y1s.append(y1)
        st1s.append(st1)

    y2s, st2s = [], []
    for h, nb in enumerate(SLICES):
        y2, st2 = _conv2(y1s[h], st1s, g1r, be1r, W2, b2r, nb)
        y2s.append(y2)
        st2s.append(st2)

    out = None
    for h, nb in enumerate(SLICES):
        out = _final(y2s[h], st2s, g2r, be2r, nb, offs[h], out)
    return out
